# Initial kernel scaffold; baseline (speedup 1.0000x reference)
import jax
import jax.numpy as jnp
from jax.experimental import pallas as pl


def kernel(feats, W0, g0, b0, W1, g1, b1, W2, g2, b2, W2t, g2t, b2t, W1t, g1t, b1t, W0t, ein0, eout0, cnt0, ein1, eout1, cnt1, ein2, eout2, cnt2, coords1, coords2):
    raise NotImplementedError("write your pallas kernel here")


# speedup vs baseline: 54.3950x; 54.3950x over previous
import functools

import jax
import jax.numpy as jnp
from jax import lax
from jax.experimental import pallas as pl
from jax.experimental.pallas import tpu as pltpu
from jax.experimental.pallas import tpu_sc as plsc

NC = 2
NS = 16
CHUNK = 128
ZBLK = 128
NW = NC * NS
ROW_ALIGN = NS * ZBLK
EDGE_ALIGN = NW * CHUNK


def _ceil_to(x, m):
    return (x + m - 1) // m * m


def _sc_edge_pass(y2d, gidx, lidx, n_acc, C):
    E_pad = gidx.shape[0]
    nch = E_pad // NW // CHUNK
    stripe = n_acc // NS
    nzb = stripe // ZBLK
    mesh = plsc.VectorSubcoreMesh(core_axis_name="c", subcore_axis_name="s")

    def body(y_hbm, g_hbm, l_hbm, out_hbm, gv, lv, rows, zbuf, acc, sem):
        c = lax.axis_index("c")
        s = lax.axis_index("s")
        w = c * NS + s

        def zrow(i, carry):
            for j in range(C // 16):
                zbuf[i, pl.ds(j * 16, 16)] = jnp.zeros((16,), jnp.float32)
            return carry

        lax.fori_loop(0, ZBLK, zrow, 0)

        def zcp(i, carry):
            pltpu.sync_copy(zbuf, acc.at[pl.ds(s * stripe + i * ZBLK, ZBLK)])
            return carry

        lax.fori_loop(0, nzb, zcp, 0)
        plsc.subcore_barrier()

        def chunk(i, carry):
            base = (w * nch + i) * CHUNK
            pltpu.sync_copy(g_hbm.at[pl.ds(base, CHUNK)], gv)
            pltpu.sync_copy(l_hbm.at[pl.ds(base, CHUNK)], lv)
            pltpu.async_copy(y_hbm.at[gv], rows, sem).wait()
            pltpu.sync_copy(rows, acc.at[lv], add=True)
            return carry

        lax.fori_loop(0, nch, chunk, 0)
        plsc.subcore_barrier()
        pltpu.sync_copy(acc.at[pl.ds(s * stripe, stripe)],
                        out_hbm.at[c, pl.ds(s * stripe, stripe)])

    f = pl.kernel(
        body,
        out_type=jax.ShapeDtypeStruct((NC, n_acc, C), jnp.float32),
        mesh=mesh,
        scratch_types=[
            pltpu.VMEM((CHUNK,), jnp.int32),
            pltpu.VMEM((CHUNK,), jnp.int32),
            pltpu.VMEM((CHUNK, C), jnp.float32),
            pltpu.VMEM((ZBLK, C), jnp.float32),
            pltpu.VMEM_SHARED((n_acc, C), jnp.float32),
            pltpu.SemaphoreType.DMA,
        ],
        compiler_params=pltpu.CompilerParams(use_tc_tiling_on_sc=False),
    )
    return f(y2d, gidx, lidx)


def _edge_prep(ein, eout, cnt, row_mul, k_mul, k_off, dummy):
    E = ein.shape[0]
    kidx = jnp.repeat(jnp.arange(27, dtype=jnp.int32), cnt,
                      total_repeat_length=E)
    gidx = ein.astype(jnp.int32) * row_mul + kidx * k_mul + k_off
    E_pad = _ceil_to(E, EDGE_ALIGN)
    pad = E_pad - E
    gidx = jnp.concatenate([gidx, jnp.zeros((pad,), jnp.int32)])
    lidx = jnp.concatenate([eout.astype(jnp.int32),
                            jnp.full((pad,), dummy, jnp.int32)])
    return gidx, lidx


def _stats(parts, n_out, blk=512):
    n_acc = parts[0].shape[1]
    cs = [p.shape[2] for p in parts]
    C = sum(cs)

    def body(*refs):
        o_ref = refs[-1]
        i = pl.program_id(0)
        xs = [r[0] + r[1] for r in refs[:-1]]
        x = xs[0] if len(xs) == 1 else jnp.concatenate(xs, axis=1)
        rows = lax.broadcasted_iota(jnp.int32, (blk, C), 0) + i * blk
        x = jnp.where(rows < n_out, x, 0.0)
        s1 = jnp.sum(x, axis=0, keepdims=True)
        s2 = jnp.sum(x * x, axis=0, keepdims=True)
        upd = jnp.concatenate([s1, s2, jnp.zeros((6, C), x.dtype)], axis=0)

        @pl.when(i == 0)
        def _init():
            o_ref[...] = upd

        @pl.when(i > 0)
        def _accum():
            o_ref[...] += upd

    return pl.pallas_call(
        body,
        grid=(n_acc // blk,),
        in_specs=[pl.BlockSpec((2, blk, c), lambda i: (0, i, 0)) for c in cs],
        out_specs=pl.BlockSpec((8, C), lambda i: (0, 0)),
        out_shape=jax.ShapeDtypeStruct((8, C), jnp.float32),
    )(*parts)


def _mm(wa, parts=None, stats=None, g=None, b=None, n=None, raw_x=None,
        skip=None, ws=None, want_act=False, blk=256):
    kdim, odim = wa.shape
    has_bn = parts is not None
    has_skip = skip is not None
    if has_bn:
        n_acc = parts[0].shape[1]
        cs = [p.shape[2] for p in parts]
        C = sum(cs)
    else:
        n_acc = raw_x.shape[0]
        C = raw_x.shape[1]

    def body(*refs):
        refs = list(refs)
        if want_act:
            act_ref = refs.pop()
        y_ref = refs.pop()
        it = iter(refs)
        if has_bn:
            p_refs = [next(it) for _ in parts]
            st_ref = next(it)
            g_ref = next(it)
            b_ref = next(it)
            xs = [r[0] + r[1] for r in p_refs]
            x = xs[0] if len(xs) == 1 else jnp.concatenate(xs, axis=1)
            m = st_ref[0:1, :] * (1.0 / n)
            var = st_ref[1:2, :] * (1.0 / n) - m * m
            inv = lax.rsqrt(var + 1e-5) * g_ref[...]
            a = jnp.maximum((x - m) * inv + b_ref[...], 0.0)
        else:
            a = next(it)[...]
        wa_ref = next(it)
        y = jnp.dot(a, wa_ref[...], preferred_element_type=jnp.float32)
        if has_skip:
            sk_ref = next(it)
            ws_ref = next(it)
            y = y + jnp.dot(sk_ref[...], ws_ref[...],
                            preferred_element_type=jnp.float32)
        y_ref[...] = y
        if want_act:
            act_ref[...] = a

    in_specs = []
    args = []
    if has_bn:
        for p, c in zip(parts, cs):
            in_specs.append(pl.BlockSpec((2, blk, c), lambda i: (0, i, 0)))
            args.append(p)
        in_specs.append(pl.BlockSpec((8, C), lambda i: (0, 0)))
        args.append(stats)
        in_specs.append(pl.BlockSpec((1, C), lambda i: (0, 0)))
        args.append(g.reshape(1, C))
        in_specs.append(pl.BlockSpec((1, C), lambda i: (0, 0)))
        args.append(b.reshape(1, C))
    else:
        in_specs.append(pl.BlockSpec((blk, C), lambda i: (i, 0)))
        args.append(raw_x)
    in_specs.append(pl.BlockSpec((kdim, odim), lambda i: (0, 0)))
    args.append(wa)
    if has_skip:
        cskip = skip.shape[1]
        in_specs.append(pl.BlockSpec((blk, cskip), lambda i: (i, 0)))
        args.append(skip)
        in_specs.append(pl.BlockSpec((cskip, odim), lambda i: (0, 0)))
        args.append(ws)

    out_specs = [pl.BlockSpec((blk, odim), lambda i: (i, 0))]
    out_shape = [jax.ShapeDtypeStruct((n_acc, odim), jnp.float32)]
    if want_act:
        out_specs.append(pl.BlockSpec((blk, C), lambda i: (i, 0)))
        out_shape.append(jax.ShapeDtypeStruct((n_acc, C), jnp.float32))

    res = pl.pallas_call(
        body,
        grid=(n_acc // blk,),
        in_specs=in_specs,
        out_specs=out_specs,
        out_shape=out_shape,
    )(*args)
    return res if want_act else res[0]


def _final_add(p, blk=512):
    n_acc, C = p.shape[1], p.shape[2]

    def body(p_ref, o_ref):
        o_ref[...] = p_ref[0] + p_ref[1]

    return pl.pallas_call(
        body,
        grid=(n_acc // blk,),
        in_specs=[pl.BlockSpec((2, blk, C), lambda i: (0, i, 0))],
        out_specs=pl.BlockSpec((blk, C), lambda i: (i, 0)),
        out_shape=jax.ShapeDtypeStruct((n_acc, C), jnp.float32),
    )(p)


def _flat_w(W):
    return W.transpose(1, 0, 2).reshape(W.shape[1], 27 * W.shape[2])


def kernel(feats, W0, g0, b0, W1, g1, b1, W2, g2, b2, W2t, g2t, b2t,
           W1t, g1t, b1t, W0t, ein0, eout0, cnt0, ein1, eout1, cnt1,
           ein2, eout2, cnt2, coords1, coords2):
    n0 = feats.shape[0]
    n1 = coords1.shape[0]
    n2 = coords2.shape[0]
    na0 = _ceil_to(n0 + 1, ROW_ALIGN)
    na1 = _ceil_to(n1 + 1, ROW_ALIGN)
    na2 = _ceil_to(n2 + 1, ROW_ALIGN)

    feats_p = jnp.pad(feats, ((0, na0 - n0), (0, 0)))
    Y0 = _mm(_flat_w(W0), raw_x=feats_p)
    gi, li = _edge_prep(ein0, eout0, cnt0, 27, 1, 0, n0)
    P0 = _sc_edge_pass(Y0.reshape(na0 * 27, 16), gi, li, na0, 16)

    st0 = _stats([P0], n0)
    Y1, act1 = _mm(_flat_w(W1), parts=[P0], stats=st0, g=g0, b=b0, n=n0,
                   want_act=True)
    gi, li = _edge_prep(ein1, eout1, cnt1, 27, 1, 0, n1)
    P1 = _sc_edge_pass(Y1.reshape(na0 * 27, 32), gi, li, na1, 32)

    st1 = _stats([P1], n1)
    Y2, act2 = _mm(_flat_w(W2), parts=[P1], stats=st1, g=g1, b=b1, n=n1,
                   want_act=True)
    gi, li = _edge_prep(ein2, eout2, cnt2, 27, 1, 0, n2)
    P2 = _sc_edge_pass(Y2.reshape(na1 * 27, 64), gi, li, na2, 64)

    st2 = _stats([P2], n2)
    Y3 = _mm(_flat_w(W2t), parts=[P2], stats=st2, g=g2, b=b2, n=n2)
    gi, li = _edge_prep(eout2, ein2, cnt2, 27, 1, 0, n1)
    P3 = _sc_edge_pass(Y3.reshape(na2 * 27, 32), gi, li, na1, 32)

    st3 = _stats([P3], n1)
    W1tf = _flat_w(W1t)
    Y4 = _mm(W1tf[:32], parts=[P3], stats=st3, g=g2t, b=b2t, n=n1,
             skip=act2, ws=W1tf[32:])
    y4r = Y4.reshape(na1 * 54, 16)
    halves = []
    for h in (0, 1):
        gi, li = _edge_prep(eout1, ein1, cnt1, 54, 2, h, n0)
        halves.append(_sc_edge_pass(y4r, gi, li, na0, 16))

    st4 = _stats(halves, n0)
    W0tp = jnp.pad(W0t, ((0, 0), (0, 0), (0, 14)))
    W0tf = _flat_w(W0tp)
    Y5 = _mm(W0tf[:32], parts=halves, stats=st4, g=g1t, b=b1t, n=n0,
             skip=act1, ws=W0tf[32:])
    gi, li = _edge_prep(ein0, eout0, cnt0, 27, 1, 0, n0)
    P5 = _sc_edge_pass(Y5.reshape(na0 * 27, 16), gi, li, na0, 16)
    out = _final_add(P5)
    return out[:n0, :2]
